# baseline (speedup 1.0000x reference)
import jax
import jax.numpy as jnp
from jax import lax
from jax.experimental import pallas as pl
from jax.experimental.pallas import tpu as pltpu
from jax.experimental.pallas import tpu_sc as plsc

B, S, D = 4, 8192, 1024
N = B * S
NC, NS = 2, 16
NW = NC * NS
N_SC = (N * 3) // 4
ROWS_PER_W = N_SC // NW
CHUNK = 8
NCH = ROWS_PER_W // CHUNK
NB = 4
LA = 2
VPR = D // 16

_STEADY_LO = NB
_STEADY_HI = ((NCH - LA) // NB) * NB


def _body(x_hbm, ids_hbm, table_hbm, out_hbm, idx_v, ibuf, gbuf,
          sems_i, sems_g, sems_o):
    wid = lax.axis_index("s") * NC + lax.axis_index("c")
    row0 = wid * ROWS_PER_W
    pltpu.sync_copy(ids_hbm.at[wid], idx_v)

    def in_copy(c, b):
        return pltpu.make_async_copy(
            x_hbm.at[pl.ds(row0 + c * CHUNK, CHUNK)], ibuf.at[b], sems_i[b])

    def g_copy(c, b):
        return pltpu.make_async_copy(
            table_hbm.at[idx_v.at[c]], gbuf.at[b], sems_g[b])

    def out_copy(c, b):
        return pltpu.make_async_copy(
            gbuf.at[b], out_hbm.at[pl.ds(row0 + c * CHUNK, CHUNK)], sems_o[b])

    def issue(c, b):
        in_copy(c, b).start()
        g_copy(c, b).start()

    def compute(b):
        def row_add(r, _):
            for j in range(VPR):
                sl = pl.ds(j * 16, 16)
                gbuf[b, r, sl] = gbuf[b, r, sl] + ibuf[b, r, sl]
            return _

        lax.fori_loop(0, CHUNK, row_add, None)

    def sub_step(c, b, la_valid, la_wait):
        in_copy(c, b).wait()
        g_copy(c, b).wait()
        if la_valid:
            la, bq = c + LA, (b + LA) % NB
            if la_wait:
                out_copy(0, bq).wait()
            issue(la, bq)
        compute(b)
        out_copy(c, b).start()

    for c in range(LA):
        issue(c, c % NB)

    for c in range(_STEADY_LO):
        sub_step(c, c % NB, True, c + LA >= NB)

    def group(g, _):
        c0 = g * NB
        for b in range(NB):
            sub_step(c0 + b, b, True, True)
        return _

    lax.fori_loop(_STEADY_LO // NB, _STEADY_HI // NB, group, None)

    for c in range(_STEADY_HI, NCH):
        sub_step(c, c % NB, c + LA < NCH, True)

    for b in range(NB):
        out_copy(0, b).wait()


@jax.jit
def _pos_embed(x, ids, table):
    mesh = plsc.VectorSubcoreMesh(
        core_axis_name="c", subcore_axis_name="s", num_cores=NC, num_subcores=NS
    )
    return pl.kernel(
        _body,
        out_type=jax.ShapeDtypeStruct((N, D), jnp.float32),
        mesh=mesh,
        scratch_types=[
            pltpu.VMEM((NCH, CHUNK), jnp.int32),
            pltpu.VMEM((NB, CHUNK, D), jnp.float32),
            pltpu.VMEM((NB, CHUNK, D), jnp.float32),
            [pltpu.SemaphoreType.DMA] * NB,
            [pltpu.SemaphoreType.DMA] * NB,
            [pltpu.SemaphoreType.DMA] * NB,
        ],
    )(x, ids, table)


def kernel(input_ids, position_ids, pos_table):
    x = input_ids.reshape(N, D)
    ids_all = position_ids.astype(jnp.int32).reshape(N)
    ids = ids_all[:N_SC].reshape(NW, NCH, CHUNK)
    out_sc = _pos_embed(x, ids, pos_table)
    tc_part = jnp.take(pos_table, ids_all[N_SC:], axis=0) + x[N_SC:]
    out = lax.dynamic_update_slice(out_sc, tc_part, (N_SC, 0))
    return out.reshape(B, S, D)


# speedup vs baseline: 1.1739x; 1.1739x over previous
import jax
import jax.numpy as jnp
from jax import lax
from jax.experimental import pallas as pl
from jax.experimental.pallas import tpu as pltpu
from jax.experimental.pallas import tpu_sc as plsc

B, S, D = 4, 8192, 1024
N = B * S
NC, NS = 2, 16
NW = NC * NS
ROWS_PER_W = N // NW
CHUNK = 8
NCH = ROWS_PER_W // CHUNK
NB = 5
LA = 3
VPR = D // 16

_STEADY_LO = NB
_STEADY_HI = ((NCH - LA) // NB) * NB


def _body(x_hbm, ids_hbm, table_hbm, out_hbm, idx_v, ibuf, gbuf,
          sems_i, sems_g, sems_o):
    wid = lax.axis_index("s") * NC + lax.axis_index("c")
    row0 = wid * ROWS_PER_W
    pltpu.sync_copy(ids_hbm.at[wid], idx_v)

    def in_copy(c, b):
        return pltpu.make_async_copy(
            x_hbm.at[pl.ds(row0 + c * CHUNK, CHUNK)], ibuf.at[b], sems_i[b])

    def g_copy(c, b):
        return pltpu.make_async_copy(
            table_hbm.at[idx_v.at[c]], gbuf.at[b], sems_g[b])

    def out_copy(c, b):
        return pltpu.make_async_copy(
            gbuf.at[b], out_hbm.at[pl.ds(row0 + c * CHUNK, CHUNK)], sems_o[b])

    def issue(c, b):
        in_copy(c, b).start()
        g_copy(c, b).start()

    def compute(b):
        def row_add(r, _):
            for j in range(VPR):
                sl = pl.ds(j * 16, 16)
                gbuf[b, r, sl] = gbuf[b, r, sl] + ibuf[b, r, sl]
            return _

        lax.fori_loop(0, CHUNK, row_add, None)

    def sub_step(c, b, la_valid, la_wait):
        in_copy(c, b).wait()
        g_copy(c, b).wait()
        if la_valid:
            la, bq = c + LA, (b + LA) % NB
            if la_wait:
                out_copy(0, bq).wait()
            issue(la, bq)
        compute(b)
        out_copy(c, b).start()

    for c in range(LA):
        issue(c, c % NB)

    for c in range(_STEADY_LO):
        sub_step(c, c % NB, True, c + LA >= NB)

    def group(g, _):
        c0 = g * NB
        for b in range(NB):
            sub_step(c0 + b, b, True, True)
        return _

    lax.fori_loop(_STEADY_LO // NB, _STEADY_HI // NB, group, None)

    for c in range(_STEADY_HI, NCH):
        sub_step(c, c % NB, c + LA < NCH, True)

    for b in range(NB):
        out_copy(0, b).wait()


@jax.jit
def _pos_embed(x, ids, table):
    mesh = plsc.VectorSubcoreMesh(
        core_axis_name="c", subcore_axis_name="s", num_cores=NC, num_subcores=NS
    )
    return pl.kernel(
        _body,
        out_type=jax.ShapeDtypeStruct((N, D), jnp.float32),
        mesh=mesh,
        scratch_types=[
            pltpu.VMEM((NCH, CHUNK), jnp.int32),
            pltpu.VMEM((NB, CHUNK, D), jnp.float32),
            pltpu.VMEM((NB, CHUNK, D), jnp.float32),
            [pltpu.SemaphoreType.DMA] * NB,
            [pltpu.SemaphoreType.DMA] * NB,
            [pltpu.SemaphoreType.DMA] * NB,
        ],
    )(x, ids, table)


def kernel(input_ids, position_ids, pos_table):
    x = input_ids.reshape(N, D)
    ids = position_ids.astype(jnp.int32).reshape(NW, NCH, CHUNK)
    out = _pos_embed(x, ids, pos_table)
    return out.reshape(B, S, D)
